# baseline (speedup 1.0000x reference)
import functools

import jax
import jax.numpy as jnp
from jax import lax
from jax.experimental import pallas as pl
from jax.experimental.pallas import tpu as pltpu
from jax.experimental.pallas import tpu_sc as plsc

NC = 2
NS = 16
NW = NC * NS
LANES = 16

BATCH = 4096
HIST = 50
EMBED_DIM = 64
NSEG = EMBED_DIM // LANES
ROWS_PER_W = BATCH // NW
NBUF = 4
NSUPER = ROWS_PER_W // NBUF


def _sc_body(table_hbm, x_hbm, scale_hbm, out_hbm,
             idx_v, scale_v,
             a0, a1, a2, a3, b0, b1, b2, b3,
             g0, g1, g2, g3, s0, s1, s2, s3, idx_sem):
    wid = lax.axis_index("s") * NC + lax.axis_index("c")
    row0 = wid * ROWS_PER_W

    pltpu.async_copy(x_hbm.at[pl.ds(row0, ROWS_PER_W)], idx_v, idx_sem)
    pltpu.sync_copy(scale_hbm, scale_v)
    svecs = [scale_v[pl.ds(c * LANES, LANES)] for c in range(NSEG)]
    pltpu.make_async_copy(x_hbm.at[pl.ds(row0, ROWS_PER_W)], idx_v, idx_sem).wait()

    A = (a0, a1, a2, a3)
    B = (b0, b1, b2, b3)
    GS = (g0, g1, g2, g3)
    SS = (s0, s1, s2, s3)

    def g_start(j, b):
        pltpu.async_copy(table_hbm.at[idx_v.at[j]], A[b], GS[b])

    def g_wait(b):
        pltpu.make_async_copy(table_hbm.at[idx_v.at[0]], A[b], GS[b]).wait()

    def s_start(j, b):
        pltpu.async_copy(B[b], out_hbm.at[row0 + j], SS[b])

    def s_wait(b):
        pltpu.make_async_copy(B[b], out_hbm.at[0], SS[b]).wait()

    for b in range(NBUF):
        g_start(b, b)

    def superstep(s, carry):
        for b in range(NBUF):
            j = s * NBUF + b
            g_wait(b)

            @pl.when(s >= 1)
            def _():
                s_wait(b)

            def row_body(r, c, b=b):
                for seg in range(NSEG):
                    B[b][r, pl.ds(seg * LANES, LANES)] = (
                        A[b][r, pl.ds(seg * LANES, LANES)] * svecs[seg]
                    )
                return c

            lax.fori_loop(0, HIST, row_body, 0, unroll=2)

            @pl.when(s < NSUPER - 1)
            def _():
                g_start(j + NBUF, b)

            s_start(j, b)
        return carry

    lax.fori_loop(0, NSUPER, superstep, 0)

    for b in range(NBUF):
        s_wait(b)


@jax.jit
def _run(table, x, scale):
    mesh = plsc.VectorSubcoreMesh(
        core_axis_name="c", subcore_axis_name="s", num_cores=NC, num_subcores=NS
    )
    f = pl.kernel(
        _sc_body,
        out_type=jax.ShapeDtypeStruct((BATCH, HIST, EMBED_DIM), jnp.float32),
        mesh=mesh,
        compiler_params=pltpu.CompilerParams(use_tc_tiling_on_sc=False),
        scratch_types=[
            pltpu.VMEM((ROWS_PER_W, HIST), jnp.int32),
            pltpu.VMEM((EMBED_DIM,), jnp.float32),
        ]
        + [pltpu.VMEM((HIST, EMBED_DIM), jnp.float32) for _ in range(2 * NBUF)]
        + [pltpu.SemaphoreType.DMA for _ in range(2 * NBUF + 1)],
    )
    return f(table, x, scale)


def kernel(x, embed_weight, p_adic_scale):
    return _run(embed_weight, x.astype(jnp.int32), p_adic_scale)
